# Initial kernel scaffold; baseline (speedup 1.0000x reference)
import jax
import jax.numpy as jnp
from jax.experimental import pallas as pl


def kernel(token_ids, weights):
    raise NotImplementedError("write your pallas kernel here")


# speedup vs baseline: 1.1029x; 1.1029x over previous
import functools

import jax
import jax.numpy as jnp
from jax import lax
from jax.experimental import pallas as pl
from jax.experimental.pallas import tpu as pltpu
from jax.experimental.pallas import tpu_sc as plsc

NUM_EMB = 1000000
DIM = 32
BATCH = 16384
SEQ = 50
B = BATCH * SEQ

NC, NS = 2, 16
NW = NC * NS
B_PER_W = B // NW
CHUNK = 1600
N_STEPS = B_PER_W // CHUNK


@functools.partial(
    pl.kernel,
    out_type=jax.ShapeDtypeStruct((B, DIM), jnp.float32),
    mesh=plsc.VectorSubcoreMesh(core_axis_name="c", subcore_axis_name="s"),
    scratch_types=[
        pltpu.VMEM((CHUNK,), jnp.int32),
        pltpu.VMEM((CHUNK, DIM), jnp.float32),
        pltpu.SemaphoreType.DMA,
    ],
    compiler_params=pltpu.CompilerParams(use_tc_tiling_on_sc=False),
)
def _embed_sc(idx_hbm, table_hbm, out_hbm, idx_v, rows_v, sem):
    wid = lax.axis_index("s") * NC + lax.axis_index("c")
    base = wid * B_PER_W

    def body(i, carry):
        off = base + i * CHUNK
        pltpu.sync_copy(idx_hbm.at[pl.ds(off, CHUNK)], idx_v)
        pltpu.async_copy(table_hbm.at[idx_v], rows_v, sem).wait()
        pltpu.sync_copy(rows_v, out_hbm.at[pl.ds(off, CHUNK)])
        return carry

    lax.fori_loop(0, N_STEPS, body, 0)


@jax.jit
def kernel(token_ids, weights):
    idx = token_ids.reshape(B).astype(jnp.int32)
    out = _embed_sc(idx, weights)
    return out.reshape(BATCH, SEQ, DIM)


# speedup vs baseline: 1.1092x; 1.0057x over previous
import functools

import jax
import jax.numpy as jnp
from jax import lax
from jax.experimental import pallas as pl
from jax.experimental.pallas import tpu as pltpu
from jax.experimental.pallas import tpu_sc as plsc

NUM_EMB = 1000000
DIM = 32
BATCH = 16384
SEQ = 50
B = BATCH * SEQ

NC, NS = 2, 16
NW = NC * NS
B_PER_W = B // NW
CHUNK = 1600
N_STEPS = B_PER_W // CHUNK


@functools.partial(
    pl.kernel,
    out_type=jax.ShapeDtypeStruct((B, DIM), jnp.float32),
    mesh=plsc.VectorSubcoreMesh(core_axis_name="c", subcore_axis_name="s"),
    scratch_types=[
        pltpu.VMEM((B_PER_W,), jnp.int32),
        pltpu.VMEM((CHUNK, DIM), jnp.float32),
        pltpu.VMEM((CHUNK, DIM), jnp.float32),
        pltpu.SemaphoreType.DMA,
        pltpu.SemaphoreType.DMA,
        pltpu.SemaphoreType.DMA,
        pltpu.SemaphoreType.DMA,
    ],
    compiler_params=pltpu.CompilerParams(use_tc_tiling_on_sc=False),
)
def _embed_sc(idx_hbm, table_hbm, out_hbm, idx_v, rows0, rows1, g0, g1, w0, w1):
    wid = lax.axis_index("s") * NC + lax.axis_index("c")
    base = wid * B_PER_W

    pltpu.sync_copy(idx_hbm.at[pl.ds(base, B_PER_W)], idx_v)

    rows = (rows0, rows1)
    gsem = (g0, g1)
    wsem = (w0, w1)

    def start_gather(step, buf, sem):
        return pltpu.async_copy(
            table_hbm.at[idx_v.at[pl.ds(step * CHUNK, CHUNK)]], buf, sem)

    g_desc = [None, None]
    w_desc = [None, None]
    g_desc[0] = start_gather(0, rows[0], gsem[0])
    for i in range(N_STEPS):
        b = i % 2
        o = (i + 1) % 2
        g_desc[b].wait()
        if i + 1 < N_STEPS:
            if w_desc[o] is not None:
                w_desc[o].wait()
            g_desc[o] = start_gather(i + 1, rows[o], gsem[o])
        w_desc[b] = pltpu.async_copy(
            rows[b], out_hbm.at[pl.ds(base + i * CHUNK, CHUNK)], wsem[b])
    w_desc[0].wait()
    w_desc[1].wait()


@jax.jit
def kernel(token_ids, weights):
    idx = token_ids.reshape(B).astype(jnp.int32)
    out = _embed_sc(idx, weights)
    return out.reshape(BATCH, SEQ, DIM)


# speedup vs baseline: 1.7823x; 1.6068x over previous
import functools

import jax
import jax.numpy as jnp
from jax import lax
from jax.experimental import pallas as pl
from jax.experimental.pallas import tpu as pltpu
from jax.experimental.pallas import tpu_sc as plsc

NUM_EMB = 1000000
DIM = 32
BATCH = 16384
SEQ = 50
B = BATCH * SEQ

NC, NS = 2, 16
NW = NC * NS
ROWS_PER_W = BATCH // NW
NB = 32
N_STEPS = ROWS_PER_W // NB
CHUNK = NB * SEQ


@functools.partial(
    pl.kernel,
    out_type=jax.ShapeDtypeStruct((BATCH, SEQ, DIM), jnp.float32),
    mesh=plsc.VectorSubcoreMesh(core_axis_name="c", subcore_axis_name="s"),
    scratch_types=[
        pltpu.VMEM((CHUNK,), jnp.int32),
        pltpu.VMEM((CHUNK, DIM), jnp.float32),
        pltpu.SemaphoreType.DMA,
        pltpu.SemaphoreType.DMA,
        pltpu.SemaphoreType.DMA,
    ],
    compiler_params=pltpu.CompilerParams(use_tc_tiling_on_sc=False),
)
def _embed_sc(idx_hbm, table_hbm, out_hbm, tok_v, rows_v, tsem, gsem, wsem):
    wid = lax.axis_index("s") * NC + lax.axis_index("c")
    base = wid * ROWS_PER_W

    def body(i, carry):
        b0 = base + i * NB
        pltpu.async_copy(
            idx_hbm.at[pl.ds(b0 * SEQ, CHUNK)], tok_v, tsem).wait()
        pltpu.async_copy(table_hbm.at[tok_v], rows_v, gsem).wait()
        descs = [
            pltpu.async_copy(
                rows_v.at[pl.ds(r * SEQ, SEQ), :], out_hbm.at[b0 + r], wsem)
            for r in range(NB)
        ]
        for d in descs:
            d.wait()
        return carry

    lax.fori_loop(0, N_STEPS, body, 0)


@jax.jit
def kernel(token_ids, weights):
    idx = token_ids.reshape(B).astype(jnp.int32)
    return _embed_sc(idx, weights)
